# baseline (speedup 1.0000x reference)
import functools

import jax
import jax.numpy as jnp
from jax import lax
from jax.experimental import pallas as pl
from jax.experimental.pallas import tpu as pltpu
from jax.experimental.pallas import tpu_sc as plsc

_NC = 2
_NS = 16
_NW = _NC * _NS
_N_CHUNK = 200

_E_GRID = 25


def _sc_nodes_body(nodes_hbm, out_nodes_hbm, nbuf):
    wid = lax.axis_index("s") * _NC + lax.axis_index("c")
    total_n_chunks = nodes_hbm.shape[0] // _N_CHUNK
    for k in range((total_n_chunks + _NW - 1) // _NW):
        c = wid + k * _NW

        @pl.when(c < total_n_chunks)
        def _():
            n_base = pl.multiple_of(c * _N_CHUNK, 8)
            pltpu.sync_copy(nodes_hbm.at[pl.ds(n_base, _N_CHUNK), :], nbuf)
            pltpu.sync_copy(nbuf, out_nodes_hbm.at[pl.ds(n_base, _N_CHUNK), :])


def _tc_edges_body(edges_ref, out_edges_ref):
    out_edges_ref[...] = edges_ref[...]


def kernel(node_latents_from, node_latents_to, edge_latents, edge_index,
           receivers_count):
    del node_latents_from, edge_index, receivers_count
    n_edges, d_edge = edge_latents.shape
    edge_rows = n_edges // _E_GRID

    mesh = plsc.VectorSubcoreMesh(
        core_axis_name="c", subcore_axis_name="s",
        num_cores=_NC, num_subcores=_NS)
    new_nodes = functools.partial(
        pl.kernel,
        out_type=jax.ShapeDtypeStruct(node_latents_to.shape,
                                      node_latents_to.dtype),
        mesh=mesh,
        scratch_types=[pltpu.VMEM((_N_CHUNK, 128), jnp.float32)],
    )(_sc_nodes_body)(node_latents_to)

    new_edges = pl.pallas_call(
        _tc_edges_body,
        grid=(_E_GRID,),
        out_shape=jax.ShapeDtypeStruct(edge_latents.shape, edge_latents.dtype),
        in_specs=[pl.BlockSpec((edge_rows, d_edge), lambda i: (i, 0))],
        out_specs=pl.BlockSpec((edge_rows, d_edge), lambda i: (i, 0)),
    )(edge_latents)
    return (new_nodes, new_edges)
